# baseline (speedup 1.0000x reference)
import jax
import jax.numpy as jnp
from jax.experimental import pallas as pl
from jax.experimental.pallas import tpu as pltpu

_C = 512
_M = 256
_BF = jnp.bfloat16
_F32 = jnp.float32

_NTAPS = (3, 3, 1, 3, 1, 4, 3, 1, 3, 1, 4, 3, 1, 3, 1, 4)
_OFFS = tuple(sum(_NTAPS[:i]) for i in range(len(_NTAPS)))
_CT_PERM = (3, 1, 2, 0)


def _dot(a, b):
    return jnp.dot(a, b, preferred_element_type=_F32)


def _shift_down(ph):
    z = jnp.zeros((1, _C), _BF)
    return jnp.concatenate([z, ph[0:_M - 1]], axis=0)


def _shift_up(ph):
    z = jnp.zeros((1, _C), _BF)
    return jnp.concatenate([ph[1:_M], z], axis=0)


def _wrows(w_ref, tap_off, ntaps):
    return w_ref[tap_off * _C:(tap_off + ntaps) * _C]


def _conv3_phases(bb, P, w_ref, li, b_ref):
    wo = _OFFS[li]
    b = b_ref[li:li + 1, :].astype(_F32)
    if P == 1:
        acc = _dot(_shift_down(bb), _wrows(w_ref, wo, 1))
        acc = acc + _dot(bb, _wrows(w_ref, wo + 1, 1))
        acc = acc + _dot(_shift_up(bb), _wrows(w_ref, wo + 2, 1))
        return [acc + b]
    down = _shift_down(bb[:, (P - 1) * _C:P * _C])
    up = _shift_up(bb[:, 0:_C])
    outs = []
    for p in range(P):
        if p == 0:
            acc = _dot(down, _wrows(w_ref, wo, 1))
            acc = acc + _dot(bb[:, 0:2 * _C], _wrows(w_ref, wo + 1, 2))
        elif p == P - 1:
            acc = _dot(bb[:, (P - 2) * _C:P * _C], _wrows(w_ref, wo, 2))
            acc = acc + _dot(up, _wrows(w_ref, wo + 2, 1))
        else:
            acc = _dot(bb[:, (p - 1) * _C:(p + 2) * _C], _wrows(w_ref, wo, 3))
        outs.append(acc + b)
    return outs


def _convt_phases(bb, P, w_ref, li, b_ref, rs=0, re=_M, full=None):
    if full is None:
        full = bb
    wo = _OFFS[li]
    b = b_ref[li:li + 1, :].astype(_F32)
    last = (P - 1) * _C
    if rs == 0:
        down = _shift_down(full[:, last:last + _C])[0:re]
    else:
        down = full[rs - 1:re - 1, last:last + _C]
    if re == _M:
        up = _shift_up(full[:, 0:_C])[rs:]
    else:
        up = full[rs + 1:re + 1, 0:_C]
    outs = []
    for p in range(P):
        x0 = bb[rs:re, p * _C:(p + 1) * _C]
        if p == 0:
            even = _dot(down, _wrows(w_ref, wo, 1)) \
                + _dot(x0, _wrows(w_ref, wo + 1, 1))
        else:
            even = _dot(bb[rs:re, (p - 1) * _C:(p + 1) * _C],
                        _wrows(w_ref, wo, 2))
        if p == P - 1:
            odd = _dot(x0, _wrows(w_ref, wo + 2, 1)) \
                + _dot(up, _wrows(w_ref, wo + 3, 1))
        else:
            odd = _dot(bb[rs:re, p * _C:(p + 2) * _C],
                       _wrows(w_ref, wo + 2, 2))
        outs.append(even + b)
        outs.append(odd + b)
    return outs


def _decoder_body(*args):
    x_ref = args[0]
    whbm = args[1:17]
    b_ref = args[17]
    out_ref = args[18]
    res_a, res_b, bb_a, bb_b, wv, stage, sems = args[19:26]

    def _issue(i):
        pltpu.make_async_copy(whbm[i], stage.at[i % 2, 0:_NTAPS[i]],
                              sems.at[i % 2]).start()

    def _arrive(i):
        pltpu.make_async_copy(whbm[i], stage.at[i % 2, 0:_NTAPS[i]],
                              sems.at[i % 2]).wait()
        perm = _CT_PERM if _NTAPS[i] == 4 else range(_NTAPS[i])
        for j, src in enumerate(perm):
            r = (_OFFS[i] + j) * _C
            wv[r:r + _C] = stage[i % 2, src].astype(_BF)
        if i + 2 < 16:
            _issue(i + 2)

    step0 = pl.program_id(0) == 0

    @pl.when(step0)
    def _():
        _issue(0)
        _issue(1)
        _arrive(0)

    xb = x_ref[0].astype(_BF)
    stem = _conv3_phases(xb, 1, wv, 0, b_ref)[0]
    res_b[:, 0:_C] = stem
    bb_b[:, 0:_C] = jnp.maximum(stem, 0.0).astype(_BF)
    res_cur, res_other = res_b, res_a
    bb_cur, bb_other = bb_b, bb_a

    P = 1
    li = 1
    for blk in range(3):
        W = P * _C
        for d in range(2):
            @pl.when(step0)
            def _(li=li):
                _arrive(li)
                _arrive(li + 1)
            accs = _conv3_phases(bb_cur[0:_M, 0:W], P, wv, li, b_ref)
            w1m = _wrows(wv, _OFFS[li + 1], 1)
            b1v = b_ref[li + 1:li + 2, :].astype(_F32)
            feeds_ct = (d == 1)
            for p in range(P):
                mid = jnp.maximum(accs[p], 0.0).astype(_BF)
                o = _dot(mid, w1m) + (b1v + res_cur[0:_M, p * _C:(p + 1) * _C])
                if feeds_ct:
                    bb_other[:, p * _C:(p + 1) * _C] = o.astype(_BF)
                else:
                    res_other[:, p * _C:(p + 1) * _C] = o
                    bb_other[:, p * _C:(p + 1) * _C] = \
                        jnp.maximum(o, 0.0).astype(_BF)
            if not feeds_ct:
                res_cur, res_other = res_other, res_cur
            bb_cur, bb_other = bb_other, bb_cur
            li += 2
        @pl.when(step0)
        def _(li=li):
            _arrive(li)
        li += 1
        if blk < 2:
            ups = _convt_phases(bb_cur[0:_M, 0:W], P, wv, li - 1, b_ref)
            for q in range(2 * P):
                res_other[:, q * _C:(q + 1) * _C] = ups[q]
                bb_other[:, q * _C:(q + 1) * _C] = \
                    jnp.maximum(ups[q], 0.0).astype(_BF)
            res_cur, res_other = res_other, res_cur
            bb_cur, bb_other = bb_other, bb_cur
        else:
            ups = _convt_phases(bb_cur[0:_M, 0:W], P, wv, li - 1, b_ref)
            inter = jnp.stack(ups, axis=0).swapaxes(0, 1).reshape(8 * _M, _C)
            out_ref[0] = inter
        P *= 2


def kernel(conv0__w, conv0__b, b0_r0_c3__w, b0_r0_c3__b, b0_r0_c1__w,
           b0_r0_c1__b, b0_r1_c3__w, b0_r1_c3__b, b0_r1_c1__w, b0_r1_c1__b,
           b0_ct__w, b0_ct__b, b1_r0_c3__w, b1_r0_c3__b, b1_r0_c1__w,
           b1_r0_c1__b, b1_r1_c3__w, b1_r1_c3__b, b1_r1_c1__w, b1_r1_c1__b,
           b1_ct__w, b1_ct__b, b2_r0_c3__w, b2_r0_c3__b, b2_r0_c1__w,
           b2_r0_c1__b, b2_r1_c3__w, b2_r1_c3__b, b2_r1_c1__w, b2_r1_c1__b,
           b2_ct__w, b2_ct__b, x):
    B = x.shape[0]
    ws = [conv0__w,
          b0_r0_c3__w, b0_r0_c1__w, b0_r1_c3__w, b0_r1_c1__w, b0_ct__w,
          b1_r0_c3__w, b1_r0_c1__w, b1_r1_c3__w, b1_r1_c1__w, b1_ct__w,
          b2_r0_c3__w, b2_r0_c1__w, b2_r1_c3__w, b2_r1_c1__w, b2_ct__w]
    bs = [conv0__b,
          b0_r0_c3__b, b0_r0_c1__b, b0_r1_c3__b, b0_r1_c1__b, b0_ct__b,
          b1_r0_c3__b, b1_r0_c1__b, b1_r1_c3__b, b1_r1_c1__b, b1_ct__b,
          b2_r0_c3__b, b2_r0_c1__b, b2_r1_c3__b, b2_r1_c1__b, b2_ct__b]
    b_all = jnp.concatenate(bs, axis=0)

    n_mm = 87
    cost = pl.CostEstimate(
        flops=2 * B * n_mm * _M * _C * _C,
        transcendentals=0,
        bytes_accessed=int(B * _M * _C * 4 + B * _M * 8 * _C * 4 + 40e6),
    )
    out = pl.pallas_call(
        _decoder_body,
        out_shape=jax.ShapeDtypeStruct((B, 8 * _M, _C), x.dtype),
        grid=(B,),
        in_specs=[pl.BlockSpec((1, _M, _C), lambda b: (b, 0, 0))]
                 + [pl.BlockSpec(memory_space=pl.ANY)] * 16
                 + [pl.BlockSpec(b_all.shape, lambda b: (0, 0))],
        out_specs=pl.BlockSpec((1, 8 * _M, _C), lambda b: (b, 0, 0)),
        scratch_shapes=[pltpu.VMEM((_M, 8 * _C), _F32),
                        pltpu.VMEM((_M, 8 * _C), _F32),
                        pltpu.VMEM((_M, 8 * _C), _BF),
                        pltpu.VMEM((_M, 8 * _C), _BF),
                        pltpu.VMEM((39 * _C, _C), _BF),
                        pltpu.VMEM((2, 4, _C, _C), _F32),
                        pltpu.SemaphoreType.DMA((2,))],
        compiler_params=pltpu.CompilerParams(
            dimension_semantics=("arbitrary",),
            vmem_limit_bytes=56 * 1024 * 1024),
        cost_estimate=cost,
    )(x, *ws, b_all)
    return out


# speedup vs baseline: 1.0821x; 1.0821x over previous
import jax
import jax.numpy as jnp
from jax.experimental import pallas as pl
from jax.experimental.pallas import tpu as pltpu

_C = 512
_M = 256
_BF = jnp.bfloat16
_F32 = jnp.float32

_NTAPS = (3, 3, 1, 3, 1, 4, 3, 1, 3, 1, 4, 3, 1, 3, 1, 4)
_OFFS = tuple(sum(_NTAPS[:i]) for i in range(len(_NTAPS)))
_CT_PERM = (3, 1, 2, 0)


def _dot(a, b):
    return jnp.dot(a, b, preferred_element_type=_F32)


def _shift_down(ph):
    z = jnp.zeros((1, _C), _BF)
    return jnp.concatenate([z, ph[0:_M - 1]], axis=0)


def _shift_up(ph):
    z = jnp.zeros((1, _C), _BF)
    return jnp.concatenate([ph[1:_M], z], axis=0)


def _wrows(w_ref, tap_off, ntaps):
    return w_ref[tap_off * _C:(tap_off + ntaps) * _C]


def _conv3_phases(bb, P, w_ref, li, b_ref):
    wo = _OFFS[li]
    b = b_ref[li:li + 1, :].astype(_F32)
    if P == 1:
        acc = _dot(_shift_down(bb), _wrows(w_ref, wo, 1))
        acc = acc + _dot(bb, _wrows(w_ref, wo + 1, 1))
        acc = acc + _dot(_shift_up(bb), _wrows(w_ref, wo + 2, 1))
        return [acc + b]
    down = _shift_down(bb[:, (P - 1) * _C:P * _C])
    up = _shift_up(bb[:, 0:_C])
    outs = []
    for p in range(P):
        if p == 0:
            acc = _dot(down, _wrows(w_ref, wo, 1))
            acc = acc + _dot(bb[:, 0:2 * _C], _wrows(w_ref, wo + 1, 2))
        elif p == P - 1:
            acc = _dot(bb[:, (P - 2) * _C:P * _C], _wrows(w_ref, wo, 2))
            acc = acc + _dot(up, _wrows(w_ref, wo + 2, 1))
        else:
            acc = _dot(bb[:, (p - 1) * _C:(p + 2) * _C], _wrows(w_ref, wo, 3))
        outs.append(acc + b)
    return outs


def _convt_phases(bb, P, w_ref, li, b_ref, rs=0, re=_M, full=None):
    if full is None:
        full = bb
    wo = _OFFS[li]
    b = b_ref[li:li + 1, :].astype(_F32)
    last = (P - 1) * _C
    if rs == 0:
        down = _shift_down(full[:, last:last + _C])[0:re]
    else:
        down = full[rs - 1:re - 1, last:last + _C]
    if re == _M:
        up = _shift_up(full[:, 0:_C])[rs:]
    else:
        up = full[rs + 1:re + 1, 0:_C]
    outs = []
    for p in range(P):
        x0 = bb[rs:re, p * _C:(p + 1) * _C]
        if p == 0:
            even = _dot(down, _wrows(w_ref, wo, 1)) \
                + _dot(x0, _wrows(w_ref, wo + 1, 1))
        else:
            even = _dot(bb[rs:re, (p - 1) * _C:(p + 1) * _C],
                        _wrows(w_ref, wo, 2))
        if p == P - 1:
            odd = _dot(x0, _wrows(w_ref, wo + 2, 1)) \
                + _dot(up, _wrows(w_ref, wo + 3, 1))
        else:
            odd = _dot(bb[rs:re, p * _C:(p + 2) * _C],
                       _wrows(w_ref, wo + 2, 2))
        outs.append(even + b)
        outs.append(odd + b)
    return outs


def _decoder_body(*args):
    x_ref = args[0]
    whbm = args[1:17]
    b_ref = args[17]
    out_ref = args[18]
    res_a, res_b, bb_a, bb_b, wv, stage, sems = args[19:26]

    @pl.when(pl.program_id(0) == 0)
    def _load_weights():
        def _issue(i):
            pltpu.make_async_copy(whbm[i], stage.at[i % 2, 0:_NTAPS[i]],
                                  sems.at[i % 2]).start()
        _issue(0)
        _issue(1)
        for i in range(16):
            pltpu.make_async_copy(whbm[i], stage.at[i % 2, 0:_NTAPS[i]],
                                  sems.at[i % 2]).wait()
            perm = _CT_PERM if _NTAPS[i] == 4 else range(_NTAPS[i])
            for j, src in enumerate(perm):
                r = (_OFFS[i] + j) * _C
                wv[r:r + _C] = stage[i % 2, src].astype(_BF)
            if i + 2 < 16:
                _issue(i + 2)

    xb = x_ref[0].astype(_BF)
    stem = _conv3_phases(xb, 1, wv, 0, b_ref)[0]
    res_b[:, 0:_C] = stem
    bb_b[:, 0:_C] = jnp.maximum(stem, 0.0).astype(_BF)
    res_cur, res_other = res_b, res_a
    bb_cur, bb_other = bb_b, bb_a

    P = 1
    li = 1
    for blk in range(3):
        W = P * _C
        for d in range(2):
            accs = _conv3_phases(bb_cur[0:_M, 0:W], P, wv, li, b_ref)
            w1m = _wrows(wv, _OFFS[li + 1], 1)
            b1v = b_ref[li + 1:li + 2, :].astype(_F32)
            feeds_ct = (d == 1)
            for p in range(P):
                mid = jnp.maximum(accs[p], 0.0).astype(_BF)
                o = _dot(mid, w1m) + (b1v + res_cur[0:_M, p * _C:(p + 1) * _C])
                if feeds_ct:
                    bb_other[:, p * _C:(p + 1) * _C] = o.astype(_BF)
                else:
                    res_other[:, p * _C:(p + 1) * _C] = o
                    bb_other[:, p * _C:(p + 1) * _C] = \
                        jnp.maximum(o, 0.0).astype(_BF)
            if not feeds_ct:
                res_cur, res_other = res_other, res_cur
            bb_cur, bb_other = bb_other, bb_cur
            li += 2
        li += 1
        if blk < 2:
            ups = _convt_phases(bb_cur[0:_M, 0:W], P, wv, li - 1, b_ref)
            for q in range(2 * P):
                res_other[:, q * _C:(q + 1) * _C] = ups[q]
                bb_other[:, q * _C:(q + 1) * _C] = \
                    jnp.maximum(ups[q], 0.0).astype(_BF)
            res_cur, res_other = res_other, res_cur
            bb_cur, bb_other = bb_other, bb_cur
        else:
            ups = _convt_phases(bb_cur[0:_M, 0:W], P, wv, li - 1, b_ref)
            inter = jnp.stack(ups, axis=0).swapaxes(0, 1).reshape(8 * _M, _C)
            out_ref[0] = inter
        P *= 2


def kernel(conv0__w, conv0__b, b0_r0_c3__w, b0_r0_c3__b, b0_r0_c1__w,
           b0_r0_c1__b, b0_r1_c3__w, b0_r1_c3__b, b0_r1_c1__w, b0_r1_c1__b,
           b0_ct__w, b0_ct__b, b1_r0_c3__w, b1_r0_c3__b, b1_r0_c1__w,
           b1_r0_c1__b, b1_r1_c3__w, b1_r1_c3__b, b1_r1_c1__w, b1_r1_c1__b,
           b1_ct__w, b1_ct__b, b2_r0_c3__w, b2_r0_c3__b, b2_r0_c1__w,
           b2_r0_c1__b, b2_r1_c3__w, b2_r1_c3__b, b2_r1_c1__w, b2_r1_c1__b,
           b2_ct__w, b2_ct__b, x):
    B = x.shape[0]
    ws = [conv0__w,
          b0_r0_c3__w, b0_r0_c1__w, b0_r1_c3__w, b0_r1_c1__w, b0_ct__w,
          b1_r0_c3__w, b1_r0_c1__w, b1_r1_c3__w, b1_r1_c1__w, b1_ct__w,
          b2_r0_c3__w, b2_r0_c1__w, b2_r1_c3__w, b2_r1_c1__w, b2_ct__w]
    bs = [conv0__b,
          b0_r0_c3__b, b0_r0_c1__b, b0_r1_c3__b, b0_r1_c1__b, b0_ct__b,
          b1_r0_c3__b, b1_r0_c1__b, b1_r1_c3__b, b1_r1_c1__b, b1_ct__b,
          b2_r0_c3__b, b2_r0_c1__b, b2_r1_c3__b, b2_r1_c1__b, b2_ct__b]
    b_all = jnp.concatenate(bs, axis=0)

    n_mm = 87
    cost = pl.CostEstimate(
        flops=2 * B * n_mm * _M * _C * _C,
        transcendentals=0,
        bytes_accessed=int(B * _M * _C * 4 + B * _M * 8 * _C * 4 + 40e6),
    )
    out = pl.pallas_call(
        _decoder_body,
        out_shape=jax.ShapeDtypeStruct((B, 8 * _M, _C), x.dtype),
        grid=(B,),
        in_specs=[pl.BlockSpec((1, _M, _C), lambda b: (b, 0, 0))]
                 + [pl.BlockSpec(memory_space=pl.ANY)] * 16
                 + [pl.BlockSpec(b_all.shape, lambda b: (0, 0))],
        out_specs=pl.BlockSpec((1, 8 * _M, _C), lambda b: (b, 0, 0)),
        scratch_shapes=[pltpu.VMEM((_M, 8 * _C), _F32),
                        pltpu.VMEM((_M, 8 * _C), _F32),
                        pltpu.VMEM((_M, 8 * _C), _BF),
                        pltpu.VMEM((_M, 8 * _C), _BF),
                        pltpu.VMEM((39 * _C, _C), _BF),
                        pltpu.VMEM((2, 4, _C, _C), _F32),
                        pltpu.SemaphoreType.DMA((2,))],
        compiler_params=pltpu.CompilerParams(
            dimension_semantics=("arbitrary",),
            vmem_limit_bytes=56 * 1024 * 1024),
        cost_estimate=cost,
    )(x, *ws, b_all)
    return out
